# baseline (speedup 1.0000x reference)
import functools

import jax
import jax.numpy as jnp
from jax import lax
from jax.experimental import pallas as pl
from jax.experimental.pallas import tpu as pltpu
import jax.experimental.pallas.tpu_sc as plsc

_Q, _N, _F = 1024, 20000, 64
_NB = 2000
_NBLK = _N // _NB
_BIGF = 3.0e38


def _argmin_body(a_ref, p_ref, a2_ref, idx_out, min_ref, idx_ref):
    nb = pl.program_id(0)
    a = a_ref[...]
    p = p_ref[...]
    pp = p * p
    eye8 = jnp.where(
        lax.broadcasted_iota(jnp.int32, (8, 3), 0)
        == lax.broadcasted_iota(jnp.int32, (8, 3), 1), 1.0, 0.0)
    rows = lax.dot_general(eye8, pp, (((1,), (1,)), ((), ())),
                           precision=lax.Precision.HIGHEST,
                           preferred_element_type=jnp.float32)
    b2 = (rows[0:1] + rows[1:2]) + rows[2:3]
    mm2 = lax.dot_general(a, p * (-2.0), (((1,), (1,)), ((), ())),
                          preferred_element_type=jnp.float32)
    d = (a2_ref[...] + b2) + mm2
    bmin = jnp.min(d, axis=1, keepdims=True)
    colf = (lax.broadcasted_iota(jnp.int32, (1, _NB), 1)
            + nb * _NB).astype(jnp.float32)
    bidx = jnp.min(
        jnp.where(d == bmin, jnp.broadcast_to(colf, d.shape), _BIGF),
        axis=1, keepdims=True)

    @pl.when(nb == 0)
    def _():
        min_ref[...] = jnp.full_like(min_ref, _BIGF)
        idx_ref[...] = jnp.zeros_like(idx_ref)

    better = bmin < min_ref[...]
    min_ref[...] = jnp.where(better, bmin, min_ref[...])
    idx_ref[...] = jnp.where(better, bidx, idx_ref[...])

    @pl.when(nb == _NBLK - 1)
    def _():
        idx_out[...] = idx_ref[...].astype(jnp.int32)


def _argmin_call(locs, pos, a2):
    return pl.pallas_call(
        _argmin_body,
        grid=(_NBLK,),
        in_specs=[
            pl.BlockSpec((_Q, 3), lambda nb: (0, 0)),
            pl.BlockSpec((_NB, 3), lambda nb: (nb, 0)),
            pl.BlockSpec((_Q, 1), lambda nb: (0, 0)),
        ],
        out_specs=pl.BlockSpec((_Q, 1), lambda nb: (0, 0)),
        out_shape=jax.ShapeDtypeStruct((_Q, 1), jnp.int32),
        scratch_shapes=[pltpu.VMEM((_Q, 1), jnp.float32),
                        pltpu.VMEM((_Q, 1), jnp.float32)],
        compiler_params=pltpu.CompilerParams(
            dimension_semantics=("arbitrary",)),
    )(locs, pos, a2)


_SC_NC, _SC_NS = 2, 16
_NW = _SC_NC * _SC_NS
_BPW = _Q // _NW


def _sc_gather(feats_l, feats_r, idx_l, idx_r):
    mesh = plsc.VectorSubcoreMesh(core_axis_name="c", subcore_axis_name="s",
                                  num_cores=_SC_NC, num_subcores=_SC_NS)

    @functools.partial(
        pl.kernel,
        out_type=[jax.ShapeDtypeStruct((_Q, _F), jnp.float32),
                  jax.ShapeDtypeStruct((_Q, _F), jnp.float32)],
        mesh=mesh,
        scratch_types=[pltpu.VMEM((_BPW,), jnp.int32),
                       pltpu.VMEM((_BPW, _F), jnp.float32),
                       pltpu.VMEM((_BPW,), jnp.int32),
                       pltpu.VMEM((_BPW, _F), jnp.float32),
                       pltpu.SemaphoreType.DMA,
                       pltpu.SemaphoreType.DMA],
        compiler_params=pltpu.CompilerParams(use_tc_tiling_on_sc=False),
    )
    def k(fl_hbm, fr_hbm, il_hbm, ir_hbm, ol_hbm, or_hbm,
          il_v, rl_v, ir_v, rr_v, sem_l, sem_r):
        wid = lax.axis_index("s") * _SC_NC + lax.axis_index("c")
        base = wid * _BPW
        pltpu.sync_copy(il_hbm.at[pl.ds(base, _BPW)], il_v)
        pltpu.sync_copy(ir_hbm.at[pl.ds(base, _BPW)], ir_v)
        cl = pltpu.async_copy(fl_hbm.at[il_v], rl_v, sem_l)
        cr = pltpu.async_copy(fr_hbm.at[ir_v], rr_v, sem_r)
        cl.wait()
        cr.wait()
        pltpu.sync_copy(rl_v, ol_hbm.at[pl.ds(base, _BPW)])
        pltpu.sync_copy(rr_v, or_hbm.at[pl.ds(base, _BPW)])

    return k(feats_l, feats_r, idx_l, idx_r)


def _mlp_body(gl_ref, gr_ref, w1a_ref, w1b_ref, b1_ref, w2_ref, b2_ref, o_ref):
    h = (jnp.dot(gl_ref[...], w1a_ref[...], preferred_element_type=jnp.float32)
         + jnp.dot(gr_ref[...], w1b_ref[...], preferred_element_type=jnp.float32)
         + b1_ref[...])
    h = jnp.maximum(h, 0.0)
    o_ref[...] = (jnp.dot(h, w2_ref[...], preferred_element_type=jnp.float32)
                  + b2_ref[...])


def _mlp_call(gl, gr, w1a, w1b, b1, w2, b2):
    return pl.pallas_call(
        _mlp_body,
        out_shape=jax.ShapeDtypeStruct((_Q, 1), jnp.float32),
    )(gl, gr, w1a, w1b, b1, w2, b2)


def kernel(locs_left, locs_right, pos_left, pos_right,
           feats_left, feats_right, W1, b1, W2, b2):
    a2l = jnp.sum(locs_left * locs_left, axis=1, keepdims=True)
    a2r = jnp.sum(locs_right * locs_right, axis=1, keepdims=True)
    idx_l = _argmin_call(locs_left, pos_left, a2l)
    idx_r = _argmin_call(locs_right, pos_right, a2r)
    gl, gr = _sc_gather(feats_left, feats_right,
                        idx_l[:, 0], idx_r[:, 0])
    out = _mlp_call(gl, gr, W1[:_F], W1[_F:], b1.reshape(1, -1),
                    W2, b2.reshape(1, 1))
    return out.reshape(-1)


# speedup vs baseline: 1.2625x; 1.2625x over previous
import functools

import jax
import jax.numpy as jnp
from jax import lax
from jax.experimental import pallas as pl
from jax.experimental.pallas import tpu as pltpu
import jax.experimental.pallas.tpu_sc as plsc

_Q, _N, _F = 1024, 20000, 64
_NB = 2048
_NPAD = 20480
_NBLK = _NPAD // _NB
_BIGF = 3.0e38


def _argmin_body(a_ref, pt_ref, a2_ref, idx_out, min_ref, idx_ref):
    nb = pl.program_id(0)
    a = a_ref[...]
    pt = pt_ref[...]
    b2 = (pt[0:1] * pt[0:1] + pt[1:2] * pt[1:2]) + pt[2:3] * pt[2:3]
    mm2 = lax.dot_general(a, pt * (-2.0), (((1,), (0,)), ((), ())),
                          preferred_element_type=jnp.float32)
    d = (a2_ref[...] + b2) + mm2
    bmin = jnp.min(d, axis=1, keepdims=True)
    colf = (lax.broadcasted_iota(jnp.int32, (1, _NB), 1)
            + nb * _NB).astype(jnp.float32)
    bidx = jnp.min(
        jnp.where(d == bmin, jnp.broadcast_to(colf, d.shape), _BIGF),
        axis=1, keepdims=True)

    @pl.when(nb == 0)
    def _():
        min_ref[...] = jnp.full_like(min_ref, _BIGF)
        idx_ref[...] = jnp.zeros_like(idx_ref)

    better = bmin < min_ref[...]
    min_ref[...] = jnp.where(better, bmin, min_ref[...])
    idx_ref[...] = jnp.where(better, bidx, idx_ref[...])

    @pl.when(nb == _NBLK - 1)
    def _():
        idx_out[...] = idx_ref[...].astype(jnp.int32)


def _argmin_call(locs, posT, a2):
    return pl.pallas_call(
        _argmin_body,
        grid=(_NBLK,),
        in_specs=[
            pl.BlockSpec((_Q, 3), lambda nb: (0, 0)),
            pl.BlockSpec((3, _NB), lambda nb: (0, nb)),
            pl.BlockSpec((_Q, 1), lambda nb: (0, 0)),
        ],
        out_specs=pl.BlockSpec((_Q, 1), lambda nb: (0, 0)),
        out_shape=jax.ShapeDtypeStruct((_Q, 1), jnp.int32),
        scratch_shapes=[pltpu.VMEM((_Q, 1), jnp.float32),
                        pltpu.VMEM((_Q, 1), jnp.float32)],
        compiler_params=pltpu.CompilerParams(
            dimension_semantics=("arbitrary",)),
    )(locs, posT, a2)


_SC_NC, _SC_NS = 2, 16
_NW = _SC_NC * _SC_NS
_BPW = _Q // _NW


def _sc_gather(feats_l, feats_r, idx_l, idx_r):
    mesh = plsc.VectorSubcoreMesh(core_axis_name="c", subcore_axis_name="s",
                                  num_cores=_SC_NC, num_subcores=_SC_NS)

    @functools.partial(
        pl.kernel,
        out_type=[jax.ShapeDtypeStruct((_Q, _F), jnp.float32),
                  jax.ShapeDtypeStruct((_Q, _F), jnp.float32)],
        mesh=mesh,
        scratch_types=[pltpu.VMEM((_BPW,), jnp.int32),
                       pltpu.VMEM((_BPW, _F), jnp.float32),
                       pltpu.VMEM((_BPW,), jnp.int32),
                       pltpu.VMEM((_BPW, _F), jnp.float32),
                       pltpu.SemaphoreType.DMA,
                       pltpu.SemaphoreType.DMA],
        compiler_params=pltpu.CompilerParams(use_tc_tiling_on_sc=False),
    )
    def k(fl_hbm, fr_hbm, il_hbm, ir_hbm, ol_hbm, or_hbm,
          il_v, rl_v, ir_v, rr_v, sem_l, sem_r):
        wid = lax.axis_index("s") * _SC_NC + lax.axis_index("c")
        base = wid * _BPW
        pltpu.sync_copy(il_hbm.at[pl.ds(base, _BPW)], il_v)
        pltpu.sync_copy(ir_hbm.at[pl.ds(base, _BPW)], ir_v)
        cl = pltpu.async_copy(fl_hbm.at[il_v], rl_v, sem_l)
        cr = pltpu.async_copy(fr_hbm.at[ir_v], rr_v, sem_r)
        cl.wait()
        cr.wait()
        pltpu.sync_copy(rl_v, ol_hbm.at[pl.ds(base, _BPW)])
        pltpu.sync_copy(rr_v, or_hbm.at[pl.ds(base, _BPW)])

    return k(feats_l, feats_r, idx_l, idx_r)


def _mlp_body(gl_ref, gr_ref, w1a_ref, w1b_ref, b1_ref, w2_ref, b2_ref, o_ref):
    h = (jnp.dot(gl_ref[...], w1a_ref[...], preferred_element_type=jnp.float32)
         + jnp.dot(gr_ref[...], w1b_ref[...], preferred_element_type=jnp.float32)
         + b1_ref[...])
    h = jnp.maximum(h, 0.0)
    o_ref[...] = (jnp.dot(h, w2_ref[...], preferred_element_type=jnp.float32)
                  + b2_ref[...])


def _mlp_call(gl, gr, w1a, w1b, b1, w2, b2):
    return pl.pallas_call(
        _mlp_body,
        out_shape=jax.ShapeDtypeStruct((_Q, 1), jnp.float32),
    )(gl, gr, w1a, w1b, b1, w2, b2)


def kernel(locs_left, locs_right, pos_left, pos_right,
           feats_left, feats_right, W1, b1, W2, b2):
    a2l = jnp.sum(locs_left * locs_left, axis=1, keepdims=True)
    a2r = jnp.sum(locs_right * locs_right, axis=1, keepdims=True)
    ptl = jnp.pad(pos_left.T, ((0, 0), (0, _NPAD - _N)),
                  constant_values=1e15)
    ptr = jnp.pad(pos_right.T, ((0, 0), (0, _NPAD - _N)),
                  constant_values=1e15)
    idx_l = _argmin_call(locs_left, ptl, a2l)
    idx_r = _argmin_call(locs_right, ptr, a2r)
    gl, gr = _sc_gather(feats_left, feats_right,
                        idx_l[:, 0], idx_r[:, 0])
    out = _mlp_call(gl, gr, W1[:_F], W1[_F:], b1.reshape(1, -1),
                    W2, b2.reshape(1, 1))
    return out.reshape(-1)
